# baseline (speedup 1.0000x reference)
import functools

import jax
import jax.numpy as jnp
from jax import lax
from jax.experimental import pallas as pl
from jax.experimental.pallas import tpu as pltpu
from jax.experimental.pallas import tpu_sc as plsc

N_TOK = 16384
DIM = 4096
LANES = 16
NC = 2
NS = 16
NW = NC * NS
ROWS_PER_W = N_TOK // NW
CHUNK = 8
N_CHUNK = ROWS_PER_W // CHUNK


def _sc_body(x_hbm, a_hbm, o_hbm, a_v, buf):
    wid = lax.axis_index("s") * NC + lax.axis_index("c")
    base = wid * ROWS_PER_W
    pltpu.sync_copy(a_hbm, a_v)

    def chunk_body(c, _):
        r0 = base + c * CHUNK
        pltpu.sync_copy(x_hbm.at[pl.ds(r0, CHUNK)], buf)

        def col_body(k, _):
            a_reg = a_v[pl.ds(k * LANES, LANES)]
            for r in range(CHUNK):
                buf[r, pl.ds(k * LANES, LANES)] = (
                    buf[r, pl.ds(k * LANES, LANES)] * a_reg
                )
            return 0

        lax.fori_loop(0, DIM // LANES, col_body, 0, unroll=2)
        pltpu.sync_copy(buf, o_hbm.at[pl.ds(r0, CHUNK)])
        return 0

    lax.fori_loop(0, N_CHUNK, chunk_body, 0)


def kernel(x, a):
    mesh = plsc.VectorSubcoreMesh(core_axis_name="c", subcore_axis_name="s")
    f = pl.kernel(
        _sc_body,
        out_type=jax.ShapeDtypeStruct((N_TOK, DIM), jnp.float32),
        mesh=mesh,
        scratch_types=[
            pltpu.VMEM((DIM,), jnp.float32),
            pltpu.VMEM((CHUNK, DIM), jnp.float32),
        ],
    )
    return f(x, a)


# speedup vs baseline: 1.5113x; 1.5113x over previous
import jax
import jax.numpy as jnp
from jax import lax
from jax.experimental import pallas as pl
from jax.experimental.pallas import tpu as pltpu
from jax.experimental.pallas import tpu_sc as plsc

N_TOK = 16384
DIM = 4096
LANES = 16
NC = 2
NS = 16
NW = NC * NS

SC_ROWS = 4096
TC_ROWS = N_TOK - SC_ROWS
ROWS_PER_W = SC_ROWS // NW
CHUNK = 8
N_CHUNK = ROWS_PER_W // CHUNK

TC_BLOCK = 512


def _sc_body(x_hbm, a_hbm, o_hbm, a_v, buf):
    wid = lax.axis_index("s") * NC + lax.axis_index("c")
    base = wid * ROWS_PER_W
    pltpu.sync_copy(a_hbm, a_v)

    def chunk_body(c, _):
        r0 = base + c * CHUNK
        pltpu.sync_copy(x_hbm.at[pl.ds(r0, CHUNK)], buf)

        def col_body(k, _):
            a_reg = a_v[pl.ds(k * LANES, LANES)]
            for r in range(CHUNK):
                buf[r, pl.ds(k * LANES, LANES)] = (
                    buf[r, pl.ds(k * LANES, LANES)] * a_reg
                )
            return 0

        lax.fori_loop(0, DIM // LANES, col_body, 0, unroll=2)
        pltpu.sync_copy(buf, o_hbm.at[pl.ds(r0, CHUNK)])
        return 0

    lax.fori_loop(0, N_CHUNK, chunk_body, 0)


def _tc_body(a_ref, x_ref, o_ref):
    o_ref[...] = x_ref[...] * a_ref[...]


def kernel(x, a):
    mesh = plsc.VectorSubcoreMesh(core_axis_name="c", subcore_axis_name="s")
    sc_out = pl.kernel(
        _sc_body,
        out_type=jax.ShapeDtypeStruct((SC_ROWS, DIM), jnp.float32),
        mesh=mesh,
        scratch_types=[
            pltpu.VMEM((DIM,), jnp.float32),
            pltpu.VMEM((CHUNK, DIM), jnp.float32),
        ],
    )(x, a)

    a2 = a.reshape(1, DIM)
    tc_out = pl.pallas_call(
        _tc_body,
        grid=(TC_ROWS // TC_BLOCK,),
        in_specs=[
            pl.BlockSpec((1, DIM), lambda i: (0, 0)),
            pl.BlockSpec((TC_BLOCK, DIM), lambda i: (SC_ROWS // TC_BLOCK + i, 0)),
        ],
        out_specs=pl.BlockSpec((TC_BLOCK, DIM), lambda i: (i, 0)),
        out_shape=jax.ShapeDtypeStruct((TC_ROWS, DIM), jnp.float32),
        compiler_params=pltpu.CompilerParams(
            dimension_semantics=("arbitrary",),
        ),
    )(a2, x)

    return jnp.concatenate([sc_out, tc_out], axis=0)


# speedup vs baseline: 3.1791x; 2.1035x over previous
import jax
import jax.numpy as jnp
from jax.experimental import pallas as pl
from jax.experimental.pallas import tpu as pltpu

N_TOK = 16384
DIM = 4096
BLOCK_ROWS = 256


def _scale_body(a_ref, x_ref, o_ref):
    o_ref[...] = x_ref[...] * a_ref[...]


def kernel(x, a):
    a2 = a.reshape(1, DIM)
    grid = (N_TOK // BLOCK_ROWS,)
    return pl.pallas_call(
        _scale_body,
        grid=grid,
        in_specs=[
            pl.BlockSpec((1, DIM), lambda i: (0, 0)),
            pl.BlockSpec((BLOCK_ROWS, DIM), lambda i: (i, 0)),
        ],
        out_specs=pl.BlockSpec((BLOCK_ROWS, DIM), lambda i: (i, 0)),
        out_shape=jax.ShapeDtypeStruct((N_TOK, DIM), jnp.float32),
        compiler_params=pltpu.CompilerParams(
            dimension_semantics=("arbitrary",),
        ),
    )(a2, x)
